# Initial kernel scaffold; baseline (speedup 1.0000x reference)
import jax
import jax.numpy as jnp
from jax.experimental import pallas as pl


def kernel(gates, zsign, mask):
    raise NotImplementedError("write your pallas kernel here")


# speedup vs baseline: 12.9517x; 12.9517x over previous
import jax
import jax.numpy as jnp
from jax.experimental import pallas as pl
from jax.experimental.pallas import tpu as pltpu

NPAD = 128
SUBLANES = 8


def _qnet_body(g_ref, zsign_ref, mask_ref, out_ref):
    d = zsign_ref.shape[0]
    v = g_ref[:, 0, 0, :]
    pr = v[:, :d]
    pi = v[:, d:]
    prob = pr * pr + pi * pi
    ev = jnp.dot(prob, zsign_ref[...],
                 preferred_element_type=jnp.float32)
    out_ref[:, 0, :] = mask_ref[:, 0, :] * (ev + 1.0) * 0.5


def kernel(gates, zsign, mask):
    B, NG, D, D2 = gates.shape
    BB = 128
    B_pad = -(-B // BB) * BB
    if B_pad != B:
        gates = jnp.pad(gates, ((0, B_pad - B), (0, 0), (0, 0), (0, 0)))
        mask = jnp.pad(mask, ((0, B_pad - B), (0, 0), (0, 0)))

    out = pl.pallas_call(
        _qnet_body,
        out_shape=jax.ShapeDtypeStruct((B_pad, 1, NPAD), jnp.float32),
        grid=(B_pad // BB,),
        in_specs=[
            pl.BlockSpec((BB, NG, SUBLANES, D2), lambda i: (i, 0, 0, 0)),
            pl.BlockSpec((D, NPAD), lambda i: (0, 0)),
            pl.BlockSpec((BB, 1, NPAD), lambda i: (i, 0, 0)),
        ],
        out_specs=pl.BlockSpec((BB, 1, NPAD), lambda i: (i, 0, 0)),
        compiler_params=pltpu.CompilerParams(
            dimension_semantics=("parallel",)),
    )(gates, zsign, mask)
    return out[:B]
